# Initial kernel scaffold; baseline (speedup 1.0000x reference)
import jax
import jax.numpy as jnp
from jax.experimental import pallas as pl


def kernel(predicted_trajectory):
    raise NotImplementedError("write your pallas kernel here")


# speedup vs baseline: 2.0651x; 2.0651x over previous
import jax
import jax.numpy as jnp
from jax.experimental import pallas as pl

_B, _C, _T = 64, 64, 80
_EPS = 1e-9


def _diversity_kernel(x_ref, y_ref, out_ref):
    x = x_ref[...]
    y = y_ref[...]
    acc = None
    for k in range(1, _C // 2):
        xs = jnp.concatenate([x[k:], x[:k]], axis=0)
        ys = jnp.concatenate([y[k:], y[:k]], axis=0)
        dx = x - xs
        dy = y - ys
        d = jnp.sqrt(dx * dx + dy * dy + _EPS)
        acc = d if acc is None else acc + d
    k = _C // 2
    xs = jnp.concatenate([x[k:], x[:k]], axis=0)
    ys = jnp.concatenate([y[k:], y[:k]], axis=0)
    dx = x - xs
    dy = y - ys
    half = jnp.sqrt(dx * dx + dy * dy + _EPS)
    total = 2.0 * jnp.sum(acc) + jnp.sum(half)
    scale = -1.0 / (_T * _C * (_C - 1) * _B)
    out_ref[...] = jnp.reshape(total * scale, (1, 1))


def kernel(predicted_trajectory):
    traj = predicted_trajectory[..., :2]
    x = traj[..., 0].transpose(1, 0, 2).reshape(_C, _B * _T)
    y = traj[..., 1].transpose(1, 0, 2).reshape(_C, _B * _T)
    out = pl.pallas_call(
        _diversity_kernel,
        out_shape=jax.ShapeDtypeStruct((1, 1), jnp.float32),
    )(x, y)
    return out[0, 0]


# speedup vs baseline: 4.1110x; 1.9907x over previous
import jax
import jax.numpy as jnp
from jax.experimental import pallas as pl
from jax.experimental.pallas import tpu as pltpu

_B, _C, _T = 64, 64, 80
_EPS = 1e-9
_WB = 128
_NB = (_B * _T) // _WB
_G = _C // 8


def _div_kernel(x_ref, y_ref, out_ref, xt_ref, yt_ref):
    for b in range(_B):
        xt_ref[:, b * _T:(b + 1) * _T] = x_ref[b]
        yt_ref[:, b * _T:(b + 1) * _T] = y_ref[b]

    def body(c, acc):
        off = pl.multiple_of(c * _WB, _WB)
        x = xt_ref[:, pl.ds(off, _WB)]
        y = yt_ref[:, pl.ds(off, _WB)]
        for k in range(1, _C // 2 + 1):
            xs = pltpu.roll(x, _C - k, 0)
            ys = pltpu.roll(y, _C - k, 0)
            dx = x - xs
            dy = y - ys
            s = dx * dx + dy * dy + _EPS
            d = s * jax.lax.rsqrt(s)
            if k == _C // 2:
                d = 0.5 * d
            acc = acc + d
        return acc

    zero = jnp.zeros((_C, _WB), jnp.float32)
    acc = jax.lax.fori_loop(0, _NB, body, zero, unroll=False)
    total = 2.0 * jnp.sum(acc)
    scale = -1.0 / (_T * _C * (_C - 1) * _B)
    out_ref[...] = jnp.reshape(total * scale, (1, 1))


def kernel(predicted_trajectory):
    traj = predicted_trajectory[..., :2]
    x = traj[..., 0]
    y = traj[..., 1]
    out = pl.pallas_call(
        _div_kernel,
        out_shape=jax.ShapeDtypeStruct((1, 1), jnp.float32),
        scratch_shapes=[
            pltpu.VMEM((_C, _B * _T), jnp.float32),
            pltpu.VMEM((_C, _B * _T), jnp.float32),
        ],
    )(x, y)
    return out[0, 0]


# speedup vs baseline: 4.2615x; 1.0366x over previous
import jax
import jax.numpy as jnp
from jax.experimental import pallas as pl
from jax.experimental.pallas import tpu as pltpu

_B, _C, _T = 64, 64, 80
_EPS = 1e-9
_WB = 128
_NB = (_B * _T) // _WB
_G = _C // 8


def _div_kernel(x_ref, y_ref, out_ref, xt_ref, yt_ref):
    for b in range(_B):
        xt_ref[:, b * _T:(b + 1) * _T] = x_ref[b]
        yt_ref[:, b * _T:(b + 1) * _T] = y_ref[b]

    def body(c, accs):
        off = pl.multiple_of(c * _WB, _WB)
        xg = [xt_ref[pl.ds(8 * g, 8), pl.ds(off, _WB)] for g in range(_G)]
        yg = [yt_ref[pl.ds(8 * g, 8), pl.ds(off, _WB)] for g in range(_G)]
        rx = [xg] + [None] * 4
        ry = [yg] + [None] * 4

        def rolled(m):
            if rx[m] is None:
                rx[m] = [pltpu.roll(v, 8 - m, 0) for v in xg]
                ry[m] = [pltpu.roll(v, 8 - m, 0) for v in yg]
            return rx[m], ry[m]

        new = list(accs)
        for a in range(5):
            b_list = range(1, 8) if a == 0 else ((0,) if a == 4 else range(8))
            for b in b_list:
                half = (8 * a + b == _C // 2)
                for g in range(_G):
                    j = (g + b) % _G
                    m = a + (g + b) // _G
                    rxm, rym = rolled(m) if m else (xg, yg)
                    dx = xg[g] - rxm[j]
                    dy = yg[g] - rym[j]
                    s = dx * dx + dy * dy + _EPS
                    d = s * jax.lax.rsqrt(s)
                    if half:
                        d = 0.5 * d
                    new[g] = new[g] + d
        return tuple(new)

    zero = jnp.zeros((8, _WB), jnp.float32)
    accs = jax.lax.fori_loop(0, _NB, body, (zero,) * _G, unroll=False)
    total = 2.0 * sum(jnp.sum(a) for a in accs)
    scale = -1.0 / (_T * _C * (_C - 1) * _B)
    out_ref[...] = jnp.reshape(total * scale, (1, 1))


def kernel(predicted_trajectory):
    traj = predicted_trajectory[..., :2]
    x = traj[..., 0]
    y = traj[..., 1]
    out = pl.pallas_call(
        _div_kernel,
        out_shape=jax.ShapeDtypeStruct((1, 1), jnp.float32),
        scratch_shapes=[
            pltpu.VMEM((_C, _B * _T), jnp.float32),
            pltpu.VMEM((_C, _B * _T), jnp.float32),
        ],
    )(x, y)
    return out[0, 0]


# speedup vs baseline: 4.3140x; 1.0123x over previous
import jax
import jax.numpy as jnp
from jax.experimental import pallas as pl
from jax.experimental.pallas import tpu as pltpu

_B, _C, _T = 64, 64, 80
_EPS = 1e-9
_WB = 128
_NB = (_B * _T) // _WB
_G = _C // 8


def _div_kernel(x_ref, y_ref, out_ref, xt_ref, yt_ref):
    for b in range(_B):
        xt_ref[:, b * _T:(b + 1) * _T] = x_ref[b]
        yt_ref[:, b * _T:(b + 1) * _T] = y_ref[b]

    def body(c, accs):
        off = pl.multiple_of(c * _WB, _WB)
        xg = [xt_ref[pl.ds(8 * g, 8), pl.ds(off, _WB)] for g in range(_G)]
        yg = [yt_ref[pl.ds(8 * g, 8), pl.ds(off, _WB)] for g in range(_G)]
        new = list(accs)
        group = 0
        for a in range(5):
            if a == 0:
                lx, ly = xg, yg
            else:
                lx = [pltpu.roll(v, 8 - a, 0) for v in xg]
                ly = [pltpu.roll(v, 8 - a, 0) for v in yg]
            for b in range(_G):
                ds = []
                for g in range(_G):
                    delta = 8 * a - b if g < _G - b else 8 * a + 8 - b
                    if not 1 <= delta <= _C // 2:
                        continue
                    j = (g + b) % _G
                    dx = lx[g] - xg[j]
                    dy = ly[g] - yg[j]
                    s = dx * dx + dy * dy + _EPS
                    d = s * jax.lax.rsqrt(s)
                    if delta == _C // 2:
                        d = 0.5 * d
                    ds.append(d)
                while len(ds) > 1:
                    ds = [p + q for p, q in zip(ds[::2], ds[1::2])] + (
                        [ds[-1]] if len(ds) % 2 else [])
                if ds:
                    new[group] = new[group] + ds[0]
                    group ^= 1
        return tuple(new)

    zero = jnp.zeros((8, _WB), jnp.float32)
    accs = jax.lax.fori_loop(0, _NB, body, (zero, zero), unroll=2)
    total = 2.0 * sum(jnp.sum(a) for a in accs)
    scale = -1.0 / (_T * _C * (_C - 1) * _B)
    out_ref[...] = jnp.reshape(total * scale, (1, 1))


def kernel(predicted_trajectory):
    traj = predicted_trajectory[..., :2]
    x = traj[..., 0]
    y = traj[..., 1]
    out = pl.pallas_call(
        _div_kernel,
        out_shape=jax.ShapeDtypeStruct((1, 1), jnp.float32),
        scratch_shapes=[
            pltpu.VMEM((_C, _B * _T), jnp.float32),
            pltpu.VMEM((_C, _B * _T), jnp.float32),
        ],
    )(x, y)
    return out[0, 0]
